# baseline (speedup 1.0000x reference)
import functools

import jax
import jax.numpy as jnp
from jax import lax
from jax.experimental import pallas as pl
from jax.experimental.pallas import tpu as pltpu
from jax.experimental.pallas import tpu_sc as plsc

D = 64
L = 16
CHUNK = 128
NROWS = 1000000


def _body(nc, b_per_w, user1d, item1d, uid_hbm, iid_hbm, out_hbm,
          uidx_v, iidx_v, ubuf0, vbuf0, ubuf1, vbuf1, out_v, sem0, sem1):
    nchunks = b_per_w // CHUNK
    wid = lax.axis_index("s") * nc + lax.axis_index("c")
    base = wid * b_per_w

    for j in range(nchunks):
        pltpu.sync_copy(uid_hbm.at[pl.ds(base + j * CHUNK, CHUNK)], uidx_v.at[j])
        pltpu.sync_copy(iid_hbm.at[pl.ds(base + j * CHUNK, CHUNK)], iidx_v.at[j])

    bufs = ((ubuf0, vbuf0, sem0), (ubuf1, vbuf1, sem1))

    def fire(c):
        ub, vb, sem = bufs[c % 2]
        cps = []
        for d in range(D):
            plane = pl.ds(d * NROWS, NROWS)
            cps.append(pltpu.async_copy(
                user1d.at[plane].at[uidx_v.at[c]], ub.at[d], sem))
            cps.append(pltpu.async_copy(
                item1d.at[plane].at[iidx_v.at[c]], vb.at[d], sem))
        return cps

    def compute(c):
        ub, vb, _ = bufs[c % 2]

        def group(g, carry):
            acc = jnp.zeros((L,), jnp.float32)
            for d in range(D):
                acc = acc + ub[d, pl.ds(g * L, L)] * vb[d, pl.ds(g * L, L)]
            out_v[pl.ds(c * CHUNK + g * L, L)] = acc
            return carry

        lax.fori_loop(0, CHUNK // L, group, 0)

    inflight = fire(0)
    for c in range(nchunks):
        nxt = fire(c + 1) if c + 1 < nchunks else []
        for cp in inflight:
            cp.wait()
        compute(c)
        inflight = nxt

    pltpu.sync_copy(out_v, out_hbm.at[pl.ds(base, b_per_w)])


def kernel(user_table, item_table, user_ids, item_ids):
    B = user_ids.shape[0]
    info = plsc.get_sparse_core_info()
    nc, ns = info.num_cores, info.num_subcores
    nw = nc * ns
    b_per_w = B // nw
    nchunks = b_per_w // CHUNK

    user1d = user_table.T.reshape(-1)
    item1d = item_table.T.reshape(-1)

    mesh = plsc.VectorSubcoreMesh(core_axis_name="c", subcore_axis_name="s")
    k = pl.kernel(
        functools.partial(_body, nc, b_per_w),
        mesh=mesh,
        compiler_params=pltpu.CompilerParams(needs_layout_passes=False),
        out_type=jax.ShapeDtypeStruct((B,), jnp.float32),
        scratch_types=[
            pltpu.VMEM((nchunks, CHUNK), jnp.int32),
            pltpu.VMEM((nchunks, CHUNK), jnp.int32),
            pltpu.VMEM((D, CHUNK), jnp.float32),
            pltpu.VMEM((D, CHUNK), jnp.float32),
            pltpu.VMEM((D, CHUNK), jnp.float32),
            pltpu.VMEM((D, CHUNK), jnp.float32),
            pltpu.VMEM((b_per_w,), jnp.float32),
            pltpu.SemaphoreType.DMA,
            pltpu.SemaphoreType.DMA,
        ],
    )
    return k(user1d, item1d, user_ids, item_ids)
